# Initial kernel scaffold; baseline (speedup 1.0000x reference)
import jax
import jax.numpy as jnp
from jax.experimental import pallas as pl


def kernel(x, edge_index, W_emb, Weights, biases, Wlin, W_out, b_out):
    raise NotImplementedError("write your pallas kernel here")


# speedup vs baseline: 7.2984x; 7.2984x over previous
import functools

import jax
import jax.numpy as jnp
from jax import lax
from jax.experimental import pallas as pl
from jax.experimental.pallas import tpu as pltpu
from jax.experimental.pallas import tpu_sc as plsc

N = 10000
E = 320000
IN = 128
H = 256
OUT = 128
L = 3
GAMMA = 0.1
EPS = 0.1

NP = 10240
BM = 512
GRID = NP // BM
CH = 80
EPT = E // 16
NCH = EPT // CH
E2 = E // 2
DPT = E2 // 16
DNCH = DPT // CH
RPT = NP // 16

_mesh = plsc.VectorSubcoreMesh(
    core_axis_name="c", subcore_axis_name="s", num_cores=2, num_subcores=16)


@functools.partial(
    pl.kernel,
    out_type=jax.ShapeDtypeStruct((2 * NP, 128), jnp.float32),
    mesh=_mesh,
    scratch_types=[
        pltpu.VMEM_SHARED((NP, 128), jnp.float32),
        pltpu.VMEM((CH, 128), jnp.float32),
        pltpu.VMEM((CH, 128), jnp.float32),
        pltpu.VMEM((CH,), jnp.int32),
    ],
)
def _deg_sc(row_hbm, cnt_hbm, acc, obuf, zbuf, ridx):
    c = lax.axis_index("c")
    s = lax.axis_index("s")
    ones_v = jnp.ones((16,), jnp.float32)
    zero_v = jnp.zeros((16,), jnp.float32)

    def fill(i, _):
        for j in range(8):
            obuf[i, pl.ds(j * 16, 16)] = ones_v
            zbuf[i, pl.ds(j * 16, 16)] = zero_v
        return 0

    lax.fori_loop(0, CH, fill, 0)
    for k in range(RPT // CH):
        pltpu.sync_copy(zbuf, acc.at[pl.ds(s * RPT + k * CH, CH)])
    plsc.subcore_barrier()

    base0 = c * E2 + s * DPT

    def step(k, _):
        pltpu.sync_copy(row_hbm.at[pl.ds(base0 + k * CH, CH)], ridx)
        pltpu.sync_copy(obuf, acc.at[ridx], add=True)
        return 0

    lax.fori_loop(0, DNCH, step, 0)
    plsc.subcore_barrier()
    pltpu.sync_copy(acc.at[pl.ds(s * RPT, RPT)],
                    cnt_hbm.at[pl.ds(c * NP + s * RPT, RPT)])


@functools.partial(
    pl.kernel,
    out_type=jax.ShapeDtypeStruct((2 * NP, 128), jnp.float32),
    mesh=_mesh,
    scratch_types=[
        pltpu.VMEM_SHARED((NP, 128), jnp.float32),
        pltpu.VMEM((CH, 128), jnp.float32),
        pltpu.VMEM((CH, 128), jnp.float32),
        pltpu.VMEM((CH,), jnp.int32),
        pltpu.VMEM((CH,), jnp.int32),
        pltpu.VMEM((CH,), jnp.int32),
        pltpu.SemaphoreType.DMA,
    ],
)
def _scatter_sc(hs_hbm, row_hbm, col_hbm, s_hbm,
                acc, gbuf, zbuf, ridx, cidx, gidx, sem):
    c = lax.axis_index("c")
    s = lax.axis_index("s")
    zero_v = jnp.zeros((16,), jnp.float32)

    def fill(i, _):
        for j in range(8):
            zbuf[i, pl.ds(j * 16, 16)] = zero_v
        return 0

    lax.fori_loop(0, CH, fill, 0)
    for k in range(RPT // CH):
        pltpu.sync_copy(zbuf, acc.at[pl.ds(s * RPT + k * CH, CH)])
    plsc.subcore_barrier()

    base0 = s * EPT
    coff = c * NP

    def step(k, _):
        b = base0 + k * CH
        pltpu.sync_copy(row_hbm.at[pl.ds(b, CH)], ridx)
        pltpu.sync_copy(col_hbm.at[pl.ds(b, CH)], cidx)
        for i in range(CH // 16):
            gidx[pl.ds(i * 16, 16)] = ridx[pl.ds(i * 16, 16)] + coff
        pltpu.async_copy(hs_hbm.at[gidx], gbuf, sem).wait()
        pltpu.sync_copy(gbuf, acc.at[cidx], add=True)
        return 0

    lax.fori_loop(0, NCH, step, 0)
    plsc.subcore_barrier()
    pltpu.sync_copy(acc.at[pl.ds(s * RPT, RPT)],
                    s_hbm.at[pl.ds(coff + s * RPT, RPT)])


def _embed_body(x_ref, w_ref, cnt_ref, h_ref, dis_ref):
    h_ref[...] = jnp.dot(x_ref[...], w_ref[...],
                         preferred_element_type=jnp.float32)
    dis_ref[...] = lax.rsqrt(1.0 + cnt_ref[0] + cnt_ref[1])


def _embed(xp, w_embT, cnt3):
    return pl.pallas_call(
        _embed_body,
        grid=(GRID,),
        in_specs=[
            pl.BlockSpec((BM, IN), lambda i: (i, 0)),
            pl.BlockSpec((IN, H), lambda i: (0, 0)),
            pl.BlockSpec((2, BM, 128), lambda i: (0, i, 0)),
        ],
        out_specs=[
            pl.BlockSpec((BM, H), lambda i: (i, 0)),
            pl.BlockSpec((BM, 128), lambda i: (i, 0)),
        ],
        out_shape=[
            jax.ShapeDtypeStruct((NP, H), jnp.float32),
            jax.ShapeDtypeStruct((NP, 128), jnp.float32),
        ],
    )(xp, w_embT, cnt3)


def _mm_body(h_ref, d_ref, w_ref, hs_ref, hw_ref):
    mm = jnp.dot(h_ref[...], w_ref[...], preferred_element_type=jnp.float32)
    d = d_ref[...]
    hs_ref[...] = jnp.stack([d * mm[:, :128], d * mm[:, 128:256]])
    hw_ref[...] = mm[:, 256:]


def _mm(h, dis, wcat):
    return pl.pallas_call(
        _mm_body,
        grid=(GRID,),
        in_specs=[
            pl.BlockSpec((BM, H), lambda i: (i, 0)),
            pl.BlockSpec((BM, 128), lambda i: (i, 0)),
            pl.BlockSpec((H, 2 * H), lambda i: (0, 0)),
        ],
        out_specs=[
            pl.BlockSpec((2, BM, 128), lambda i: (0, i, 0)),
            pl.BlockSpec((BM, H), lambda i: (i, 0)),
        ],
        out_shape=[
            jax.ShapeDtypeStruct((2, NP, 128), jnp.float32),
            jax.ShapeDtypeStruct((NP, H), jnp.float32),
        ],
    )(h, dis, wcat)


def _comb_body(h_ref, hw_ref, hs_ref, s_ref, d_ref, b_ref, out_ref):
    d = d_ref[...]
    agg0 = d * (s_ref[0] + hs_ref[0])
    agg1 = d * (s_ref[1] + hs_ref[1])
    z = hw_ref[...] + jnp.concatenate([agg0, agg1], axis=1) + b_ref[0:1, :]
    out_ref[...] = h_ref[...] + EPS * jnp.tanh(z)


def _combine(h, hw, hs3, s3, dis, bpad):
    return pl.pallas_call(
        _comb_body,
        grid=(GRID,),
        in_specs=[
            pl.BlockSpec((BM, H), lambda i: (i, 0)),
            pl.BlockSpec((BM, H), lambda i: (i, 0)),
            pl.BlockSpec((2, BM, 128), lambda i: (0, i, 0)),
            pl.BlockSpec((2, BM, 128), lambda i: (0, i, 0)),
            pl.BlockSpec((BM, 128), lambda i: (i, 0)),
            pl.BlockSpec((8, H), lambda i: (0, 0)),
        ],
        out_specs=pl.BlockSpec((BM, H), lambda i: (i, 0)),
        out_shape=jax.ShapeDtypeStruct((NP, H), jnp.float32),
    )(h, hw, hs3, s3, dis, bpad)


def _out_body(h_ref, w_ref, b_ref, out_ref):
    out_ref[...] = jnp.dot(h_ref[...], w_ref[...],
                           preferred_element_type=jnp.float32) + b_ref[0:1, :]


def _proj_out(h, w_outT, bpad):
    return pl.pallas_call(
        _out_body,
        grid=(GRID,),
        in_specs=[
            pl.BlockSpec((BM, H), lambda i: (i, 0)),
            pl.BlockSpec((H, OUT), lambda i: (0, 0)),
            pl.BlockSpec((8, OUT), lambda i: (0, 0)),
        ],
        out_specs=pl.BlockSpec((BM, OUT), lambda i: (i, 0)),
        out_shape=jax.ShapeDtypeStruct((NP, OUT), jnp.float32),
    )(h, w_outT, bpad)


def kernel(x, edge_index, W_emb, Weights, biases, Wlin, W_out, b_out):
    row = edge_index[0]
    col = edge_index[1]

    xp = jnp.zeros((NP, IN), jnp.float32).at[:N].set(x)
    eye = jnp.eye(H, dtype=jnp.float32)
    wcats = [jnp.concatenate(
        [Wlin[l].T, Weights[l] - Weights[l].T - GAMMA * eye], axis=1)
        for l in range(L)]
    bpads = [jnp.broadcast_to(biases[l][None, :], (8, H)) for l in range(L)]
    bout_pad = jnp.broadcast_to(b_out[None, :], (8, OUT))

    cnt = _deg_sc(row)
    cnt3 = cnt.reshape(2, NP, 128)
    h, dis = _embed(xp, W_emb.T, cnt3)

    for l in range(L):
        hs3, hw = _mm(h, dis, wcats[l])
        s_flat = _scatter_sc(hs3.reshape(2 * NP, 128), row, col)
        h = _combine(h, hw, hs3, s_flat.reshape(2, NP, 128), dis, bpads[l])

    out = _proj_out(h, W_out.T, bout_pad)
    return out[:N]


# speedup vs baseline: 15.3118x; 2.0980x over previous
import functools

import jax
import jax.numpy as jnp
from jax import lax
from jax.experimental import pallas as pl
from jax.experimental.pallas import tpu as pltpu
from jax.experimental.pallas import tpu_sc as plsc

N = 10000
E = 320000
IN = 128
H = 256
OUT = 128
L = 3
GAMMA = 0.1
EPS = 0.1

NP = 10240
BM = 512
GRID = NP // BM
CH = 80
EPT = E // 16
NCH = EPT // CH
E2 = E // 2
DPT = E2 // 16
DNCH = DPT // CH
RPT = NP // 16

_mesh = plsc.VectorSubcoreMesh(
    core_axis_name="c", subcore_axis_name="s", num_cores=2, num_subcores=16)


@functools.partial(
    pl.kernel,
    out_type=jax.ShapeDtypeStruct((2 * NP, 128), jnp.float32),
    mesh=_mesh,
    scratch_types=[
        pltpu.VMEM_SHARED((NP, 128), jnp.float32),
        pltpu.VMEM((CH, 128), jnp.float32),
        pltpu.VMEM((CH, 128), jnp.float32),
        pltpu.VMEM((DNCH, CH), jnp.int32),
    ],
)
def _deg_sc(row_hbm, cnt_hbm, acc, obuf, zbuf, ridx2):
    c = lax.axis_index("c")
    s = lax.axis_index("s")
    ones_v = jnp.ones((16,), jnp.float32)
    zero_v = jnp.zeros((16,), jnp.float32)

    def fill(i, _):
        for j in range(8):
            obuf[i, pl.ds(j * 16, 16)] = ones_v
            zbuf[i, pl.ds(j * 16, 16)] = zero_v
        return 0

    lax.fori_loop(0, CH, fill, 0)
    for k in range(RPT // CH):
        pltpu.sync_copy(zbuf, acc.at[pl.ds(s * RPT + k * CH, CH)])
    pltpu.sync_copy(row_hbm.at[c * 16 + s], ridx2)
    plsc.subcore_barrier()

    def step(k, _):
        pltpu.sync_copy(obuf, acc.at[ridx2.at[k]], add=True)
        return 0

    lax.fori_loop(0, DNCH, step, 0)
    plsc.subcore_barrier()
    pltpu.sync_copy(acc.at[pl.ds(s * RPT, RPT)],
                    cnt_hbm.at[pl.ds(c * NP + s * RPT, RPT)])


NB = 25
BNC = NCH // NB


@functools.partial(
    pl.kernel,
    out_type=jax.ShapeDtypeStruct((2 * NP, 128), jnp.float32),
    mesh=_mesh,
    scratch_types=[
        pltpu.VMEM_SHARED((NP, 128), jnp.float32),
        pltpu.VMEM((CH, 128), jnp.float32),
        pltpu.VMEM((CH, 128), jnp.float32),
        pltpu.VMEM((BNC, CH), jnp.int32),
        pltpu.VMEM((BNC, CH), jnp.int32),
        pltpu.VMEM((BNC, CH), jnp.int32),
        pltpu.VMEM((BNC, CH), jnp.int32),
        pltpu.VMEM((CH,), jnp.int32),
        pltpu.VMEM((CH,), jnp.int32),
        pltpu.SemaphoreType.DMA,
        pltpu.SemaphoreType.DMA,
        pltpu.SemaphoreType.DMA,
        pltpu.SemaphoreType.DMA,
    ],
)
def _scatter_sc(hs_hbm, row_hbm, col_hbm, s_hbm,
                acc, gbuf0, gbuf1, ridxA, ridxB, cidxA, cidxB,
                gidx0, gidx1, sg0, sg1, semri, semci):
    c = lax.axis_index("c")
    s = lax.axis_index("s")
    gbuf = (gbuf0, gbuf1)
    ridx = (ridxA, ridxB)
    cidx = (cidxA, cidxB)
    gidx = (gidx0, gidx1)
    sg = (sg0, sg1)
    zero_v = jnp.zeros((16,), jnp.float32)

    def fill(i, _):
        for j in range(8):
            gbuf0[i, pl.ds(j * 16, 16)] = zero_v
        return 0

    lax.fori_loop(0, CH, fill, 0)
    for k in range(RPT // CH):
        pltpu.sync_copy(gbuf0, acc.at[pl.ds(s * RPT + k * CH, CH)])
    pltpu.sync_copy(row_hbm.at[s * NB], ridx[0])
    pltpu.sync_copy(col_hbm.at[s * NB], cidx[0])
    plsc.subcore_barrier()

    coff = c * NP

    def prep(bs, k, sl):
        for i in range(CH // 16):
            gidx[sl][pl.ds(i * 16, 16)] = (
                ridx[bs][k, pl.ds(i * 16, 16)] + coff)
        pltpu.async_copy(hs_hbm.at[gidx[sl]], gbuf[sl], sg[sl])

    def wait_gather(sl):
        pltpu.make_async_copy(hs_hbm.at[pl.ds(0, CH)], gbuf[sl], sg[sl]).wait()

    prep(0, 0, 0)
    for b in range(NB):
        bs = b % 2
        bn = (b + 1) % 2
        if b + 1 < NB:
            pltpu.async_copy(row_hbm.at[s * NB + b + 1], ridx[bn], semri)
            pltpu.async_copy(col_hbm.at[s * NB + b + 1], cidx[bn], semci)

        def step(kk, _):
            for p in range(2):
                k = kk * 2 + p
                q = 1 - p

                @pl.when(k + 1 < BNC)
                def _():
                    prep(bs, k + 1, q)

                wait_gather(p)
                pltpu.sync_copy(gbuf[p], acc.at[cidx[bs].at[k]], add=True)
            return 0

        lax.fori_loop(0, BNC // 2, step, 0)
        if b + 1 < NB:
            pltpu.make_async_copy(row_hbm.at[s * NB], ridx[bn], semri).wait()
            pltpu.make_async_copy(col_hbm.at[s * NB], cidx[bn], semci).wait()
            prep(bn, 0, 0)
    plsc.subcore_barrier()
    pltpu.sync_copy(acc.at[pl.ds(s * RPT, RPT)],
                    s_hbm.at[pl.ds(coff + s * RPT, RPT)])


def _embed_body(x_ref, w_ref, cnt_ref, h_ref, dis_ref):
    h_ref[...] = jnp.dot(x_ref[...], w_ref[...],
                         preferred_element_type=jnp.float32)
    dis_ref[...] = lax.rsqrt(1.0 + cnt_ref[0] + cnt_ref[1])


def _embed(xp, w_embT, cnt3):
    return pl.pallas_call(
        _embed_body,
        grid=(GRID,),
        in_specs=[
            pl.BlockSpec((BM, IN), lambda i: (i, 0)),
            pl.BlockSpec((IN, H), lambda i: (0, 0)),
            pl.BlockSpec((2, BM, 128), lambda i: (0, i, 0)),
        ],
        out_specs=[
            pl.BlockSpec((BM, H), lambda i: (i, 0)),
            pl.BlockSpec((BM, 128), lambda i: (i, 0)),
        ],
        out_shape=[
            jax.ShapeDtypeStruct((NP, H), jnp.float32),
            jax.ShapeDtypeStruct((NP, 128), jnp.float32),
        ],
    )(xp, w_embT, cnt3)


def _mm_body(h_ref, d_ref, w_ref, hs_ref, hw_ref):
    mm = jnp.dot(h_ref[...], w_ref[...], preferred_element_type=jnp.float32)
    d = d_ref[...]
    hs_ref[...] = jnp.stack([d * mm[:, :128], d * mm[:, 128:256]])
    hw_ref[...] = mm[:, 256:]


def _mm(h, dis, wcat):
    return pl.pallas_call(
        _mm_body,
        grid=(GRID,),
        in_specs=[
            pl.BlockSpec((BM, H), lambda i: (i, 0)),
            pl.BlockSpec((BM, 128), lambda i: (i, 0)),
            pl.BlockSpec((H, 2 * H), lambda i: (0, 0)),
        ],
        out_specs=[
            pl.BlockSpec((2, BM, 128), lambda i: (0, i, 0)),
            pl.BlockSpec((BM, H), lambda i: (i, 0)),
        ],
        out_shape=[
            jax.ShapeDtypeStruct((2, NP, 128), jnp.float32),
            jax.ShapeDtypeStruct((NP, H), jnp.float32),
        ],
    )(h, dis, wcat)


def _comb_body(h_ref, hw_ref, hs_ref, s_ref, d_ref, b_ref, out_ref):
    d = d_ref[...]
    agg0 = d * (s_ref[0] + hs_ref[0])
    agg1 = d * (s_ref[1] + hs_ref[1])
    z = hw_ref[...] + jnp.concatenate([agg0, agg1], axis=1) + b_ref[0:1, :]
    out_ref[...] = h_ref[...] + EPS * jnp.tanh(z)


def _combine(h, hw, hs3, s3, dis, bpad):
    return pl.pallas_call(
        _comb_body,
        grid=(GRID,),
        in_specs=[
            pl.BlockSpec((BM, H), lambda i: (i, 0)),
            pl.BlockSpec((BM, H), lambda i: (i, 0)),
            pl.BlockSpec((2, BM, 128), lambda i: (0, i, 0)),
            pl.BlockSpec((2, BM, 128), lambda i: (0, i, 0)),
            pl.BlockSpec((BM, 128), lambda i: (i, 0)),
            pl.BlockSpec((8, H), lambda i: (0, 0)),
        ],
        out_specs=pl.BlockSpec((BM, H), lambda i: (i, 0)),
        out_shape=jax.ShapeDtypeStruct((NP, H), jnp.float32),
    )(h, hw, hs3, s3, dis, bpad)


def _out_body(h_ref, w_ref, b_ref, out_ref):
    out_ref[...] = jnp.dot(h_ref[...], w_ref[...],
                           preferred_element_type=jnp.float32) + b_ref[0:1, :]


def _proj_out(h, w_outT, bpad):
    return pl.pallas_call(
        _out_body,
        grid=(GRID,),
        in_specs=[
            pl.BlockSpec((BM, H), lambda i: (i, 0)),
            pl.BlockSpec((H, OUT), lambda i: (0, 0)),
            pl.BlockSpec((8, OUT), lambda i: (0, 0)),
        ],
        out_specs=pl.BlockSpec((BM, OUT), lambda i: (i, 0)),
        out_shape=jax.ShapeDtypeStruct((NP, OUT), jnp.float32),
    )(h, w_outT, bpad)


def kernel(x, edge_index, W_emb, Weights, biases, Wlin, W_out, b_out):
    row = edge_index[0]
    col = edge_index[1]
    row_deg = row.reshape(32, DNCH, CH)
    row_sc = row.reshape(16 * NB, BNC, CH)
    col_sc = col.reshape(16 * NB, BNC, CH)

    xp = jnp.zeros((NP, IN), jnp.float32).at[:N].set(x)
    eye = jnp.eye(H, dtype=jnp.float32)
    wcats = [jnp.concatenate(
        [Wlin[l].T, Weights[l] - Weights[l].T - GAMMA * eye], axis=1)
        for l in range(L)]
    bpads = [jnp.broadcast_to(biases[l][None, :], (8, H)) for l in range(L)]
    bout_pad = jnp.broadcast_to(b_out[None, :], (8, OUT))

    cnt = _deg_sc(row_deg)
    cnt3 = cnt.reshape(2, NP, 128)
    h, dis = _embed(xp, W_emb.T, cnt3)

    for l in range(L):
        hs3, hw = _mm(h, dis, wcats[l])
        s_flat = _scatter_sc(hs3.reshape(2 * NP, 128), row_sc, col_sc)
        h = _combine(h, hw, hs3, s_flat.reshape(2, NP, 128), dis, bpads[l])

    out = _proj_out(h, W_out.T, bout_pad)
    return out[:N]
